# baseline (speedup 1.0000x reference)
import jax
import jax.numpy as jnp
from jax import lax
from jax.experimental import pallas as pl
from jax.experimental.pallas import tpu as pltpu
from jax.experimental.pallas import tpu_sc as plsc

_INFO = plsc.get_sparse_core_info()
_NC, _NS, _L = _INFO.num_cores, _INFO.num_subcores, _INFO.num_lanes
_NW = _NC * _NS
_B = 16384
_K = 16
_V = 1000000
_BPW = _B // _NW
_CH = 128
_NCH = _BPW // _CH
_BLK = _BPW // _L

_W = 1024
_GRID = -(-_V // _W)
_RB = _GRID * 8
_FLAT = _RB * _K * 128


def _permute_body(x_ref, o_ref):
    x = x_ref[...]
    o_ref[...] = jnp.transpose(x.reshape(_K, 8, 128), (1, 0, 2))


def _permute(t):
    return pl.pallas_call(
        _permute_body,
        grid=(_GRID,),
        in_specs=[pl.BlockSpec((_K, _W), lambda c: (0, c))],
        out_specs=pl.BlockSpec((8, _K, 128), lambda c: (c, 0, 0)),
        out_shape=jax.ShapeDtypeStruct((_RB, _K, 128), jnp.float32),
    )(t)


def _fm_body(user_hbm, product_hbm, uf_hbm, pf_hbm, ub_hbm, pb_hbm, bias_hbm,
             out_hbm, idx_u2, idx_p2, idx_uf, idx_pf, ubuf, pbuf, ub_v, pb_v,
             bias_v, out_v, dummy, sem, sem2):
    wid = lax.axis_index("s") * _NC + lax.axis_index("c")
    base = wid * _BPW

    for j in range(_NCH):
        pltpu.sync_copy(user_hbm.at[pl.ds(base + j * _CH, _CH)], idx_u2.at[j])
        pltpu.sync_copy(product_hbm.at[pl.ds(base + j * _CH, _CH)], idx_p2.at[j])
    pltpu.sync_copy(user_hbm.at[pl.ds(base, _BPW)], idx_uf)
    pltpu.sync_copy(product_hbm.at[pl.ds(base, _BPW)], idx_pf)
    pltpu.sync_copy(bias_hbm, bias_v)

    bcopies = []
    for j in range(_NCH):
        sl = pl.ds(j * _CH, _CH)
        bcopies.append(pltpu.async_copy(ub_hbm.at[idx_u2.at[j]], ub_v.at[sl], sem2))
        bcopies.append(pltpu.async_copy(pb_hbm.at[idx_p2.at[j]], pb_v.at[sl], sem2))

    def fire_u(b, carry):
        r = idx_uf[pl.ds(b * _L, _L)]
        off = ((r >> 7) << 11) + (r & 127)
        for k in range(_K):
            pltpu.async_copy(uf_hbm.at[off + (k * 128)],
                             ubuf.at[k, pl.ds(b * _L, _L)], sem)
        return carry

    def fire_p(b, carry):
        r = idx_pf[pl.ds(b * _L, _L)]
        off = ((r >> 7) << 11) + (r & 127)
        for k in range(_K):
            pltpu.async_copy(pf_hbm.at[off + (k * 128)],
                             pbuf.at[k, pl.ds(b * _L, _L)], sem)
        return carry

    lax.fori_loop(0, _BLK, fire_u, 0)
    lax.fori_loop(0, _BLK, fire_p, 0)

    pltpu.make_async_copy(uf_hbm.at[pl.ds(0, 2 * _BPW * _K)], dummy, sem).wait()
    for c in bcopies:
        c.wait()

    bias_vec = bias_v[...]

    def block(b, carry):
        sl = pl.ds(b * _L, _L)
        acc = bias_vec + ub_v[sl] + pb_v[sl]
        for k in range(_K):
            acc = acc + ubuf[k, sl] * pbuf[k, sl]
        out_v[sl] = acc
        return carry

    lax.fori_loop(0, _BLK, block, 0)
    pltpu.sync_copy(out_v, out_hbm.at[pl.ds(base, _BPW)])


@jax.jit
def kernel(user, product, user_embeds, products_embeds, user_bias,
           products_bias, bias):
    uf = _permute(user_embeds.T).reshape(_FLAT)
    pf = _permute(products_embeds.T).reshape(_FLAT)
    bias16 = jnp.broadcast_to(bias, (_L,))
    f = pl.kernel(
        _fm_body,
        out_type=jax.ShapeDtypeStruct((_B,), jnp.float32),
        mesh=plsc.VectorSubcoreMesh(core_axis_name="c", subcore_axis_name="s"),
        compiler_params=pltpu.CompilerParams(use_tc_tiling_on_sc=False,
                                             needs_layout_passes=False),
        scratch_types=[
            pltpu.VMEM((_NCH, _CH), jnp.int32),
            pltpu.VMEM((_NCH, _CH), jnp.int32),
            pltpu.VMEM((_BPW,), jnp.int32),
            pltpu.VMEM((_BPW,), jnp.int32),
            pltpu.VMEM((_K, _BPW), jnp.float32),
            pltpu.VMEM((_K, _BPW), jnp.float32),
            pltpu.VMEM((_BPW,), jnp.float32),
            pltpu.VMEM((_BPW,), jnp.float32),
            pltpu.VMEM((_L,), jnp.float32),
            pltpu.VMEM((_BPW,), jnp.float32),
            pltpu.VMEM((2 * _BPW * _K,), jnp.float32),
            pltpu.SemaphoreType.DMA,
            pltpu.SemaphoreType.DMA,
        ],
    )
    return f(user, product, uf, pf, user_bias.reshape(-1),
             products_bias.reshape(-1), bias16)


# speedup vs baseline: 3.6169x; 3.6169x over previous
import jax
import jax.numpy as jnp
from jax import lax
from jax.experimental import pallas as pl
from jax.experimental.pallas import tpu as pltpu
from jax.experimental.pallas import tpu_sc as plsc

_INFO = plsc.get_sparse_core_info()
_NC, _NS, _L = _INFO.num_cores, _INFO.num_subcores, _INFO.num_lanes
_NW = _NC * _NS
_B = 16384
_K = 16
_V = 1000000
_BPW = _B // _NW
_CH = 128
_NCH = _BPW // _CH
_BLK = _BPW // _L

_W = 8192
_GRID = -(-_V // _W)
_WB = _W // 128
_RBW = _GRID * _WB
_STRIDE = _RBW * 128
_FLAT = _K * _STRIDE


def _permute_body(x_ref, o_ref):
    o_ref[...] = x_ref[...].reshape(2, 8, _WB, 128)


def _permute(t):
    return pl.pallas_call(
        _permute_body,
        grid=(_GRID,),
        in_specs=[pl.BlockSpec((_K, _W), lambda c: (0, c))],
        out_specs=pl.BlockSpec((2, 8, _WB, 128), lambda c: (0, 0, c, 0)),
        out_shape=jax.ShapeDtypeStruct((2, 8, _RBW, 128), jnp.float32),
    )(t)


def _fm_body(user_hbm, product_hbm, uf_hbm, pf_hbm, ub_hbm, pb_hbm, bias_hbm,
             out_hbm, idx_u2, idx_p2, idx_uf, idx_pf, ubuf, pbuf, ub_v, pb_v,
             bias_v, out_v, dummy, sem, sem2):
    wid = lax.axis_index("s") * _NC + lax.axis_index("c")
    base = wid * _BPW

    for j in range(_NCH):
        pltpu.sync_copy(user_hbm.at[pl.ds(base + j * _CH, _CH)], idx_u2.at[j])
        pltpu.sync_copy(product_hbm.at[pl.ds(base + j * _CH, _CH)], idx_p2.at[j])
    pltpu.sync_copy(user_hbm.at[pl.ds(base, _BPW)], idx_uf)
    pltpu.sync_copy(product_hbm.at[pl.ds(base, _BPW)], idx_pf)
    pltpu.sync_copy(bias_hbm, bias_v)

    bcopies = []
    for j in range(_NCH):
        sl = pl.ds(j * _CH, _CH)
        bcopies.append(pltpu.async_copy(ub_hbm.at[idx_u2.at[j]], ub_v.at[sl], sem2))
        bcopies.append(pltpu.async_copy(pb_hbm.at[idx_p2.at[j]], pb_v.at[sl], sem2))

    def fire_u(b, carry):
        r = idx_uf[pl.ds(b * _L, _L)]
        for k in range(_K):
            pltpu.async_copy(uf_hbm.at[r + (k * _STRIDE)],
                             ubuf.at[k, pl.ds(b * _L, _L)], sem)
        return carry

    def fire_p(b, carry):
        r = idx_pf[pl.ds(b * _L, _L)]
        for k in range(_K):
            pltpu.async_copy(pf_hbm.at[r + (k * _STRIDE)],
                             pbuf.at[k, pl.ds(b * _L, _L)], sem)
        return carry

    lax.fori_loop(0, _BLK, fire_u, 0)
    lax.fori_loop(0, _BLK, fire_p, 0)

    pltpu.make_async_copy(uf_hbm.at[pl.ds(0, 2 * _BPW * _K)], dummy, sem).wait()
    for c in bcopies:
        c.wait()

    bias_vec = bias_v[...]

    def block(b, carry):
        sl = pl.ds(b * _L, _L)
        acc = bias_vec + ub_v[sl] + pb_v[sl]
        for k in range(_K):
            acc = acc + ubuf[k, sl] * pbuf[k, sl]
        out_v[sl] = acc
        return carry

    lax.fori_loop(0, _BLK, block, 0)
    pltpu.sync_copy(out_v, out_hbm.at[pl.ds(base, _BPW)])


@jax.jit
def kernel(user, product, user_embeds, products_embeds, user_bias,
           products_bias, bias):
    uf = _permute(user_embeds.T).reshape(_FLAT)
    pf = _permute(products_embeds.T).reshape(_FLAT)
    bias16 = jnp.broadcast_to(bias, (_L,))
    f = pl.kernel(
        _fm_body,
        out_type=jax.ShapeDtypeStruct((_B,), jnp.float32),
        mesh=plsc.VectorSubcoreMesh(core_axis_name="c", subcore_axis_name="s"),
        compiler_params=pltpu.CompilerParams(use_tc_tiling_on_sc=False,
                                             needs_layout_passes=False),
        scratch_types=[
            pltpu.VMEM((_NCH, _CH), jnp.int32),
            pltpu.VMEM((_NCH, _CH), jnp.int32),
            pltpu.VMEM((_BPW,), jnp.int32),
            pltpu.VMEM((_BPW,), jnp.int32),
            pltpu.VMEM((_K, _BPW), jnp.float32),
            pltpu.VMEM((_K, _BPW), jnp.float32),
            pltpu.VMEM((_BPW,), jnp.float32),
            pltpu.VMEM((_BPW,), jnp.float32),
            pltpu.VMEM((_L,), jnp.float32),
            pltpu.VMEM((_BPW,), jnp.float32),
            pltpu.VMEM((2 * _BPW * _K,), jnp.float32),
            pltpu.SemaphoreType.DMA,
            pltpu.SemaphoreType.DMA,
        ],
    )
    return f(user, product, uf, pf, user_bias.reshape(-1),
             products_bias.reshape(-1), bias16)


# speedup vs baseline: 5.2941x; 1.4637x over previous
import jax
import jax.numpy as jnp
from jax import lax
from jax.experimental import pallas as pl
from jax.experimental.pallas import tpu as pltpu
from jax.experimental.pallas import tpu_sc as plsc

_INFO = plsc.get_sparse_core_info()
_NC, _NS, _L = _INFO.num_cores, _INFO.num_subcores, _INFO.num_lanes
_NW = _NC * _NS
_B = 16384
_K = 16
_V = 1000000
_BPW = _B // _NW
_CH = 128
_NCH = _BPW // _CH
_BLK = _BPW // _L

_W = 32768
_GRID = -(-_V // _W)
_WB = _W // 128
_RBW = _GRID * _WB
_STRIDE = _RBW * 128
_FLAT = _K * _STRIDE


def _detile_body(x_ref, y_ref, ox_ref, oy_ref):
    ox_ref[...] = x_ref[...].reshape(2, 8, _WB, 128)
    oy_ref[...] = y_ref[...].reshape(2, 8, _WB, 128)


def _detile(tu, tp):
    spec_in = pl.BlockSpec((_K, _W), lambda c: (0, c))
    spec_out = pl.BlockSpec((2, 8, _WB, 128), lambda c: (0, 0, c, 0))
    oshape = jax.ShapeDtypeStruct((2, 8, _RBW, 128), jnp.float32)
    return pl.pallas_call(
        _detile_body,
        grid=(_GRID,),
        in_specs=[spec_in, spec_in],
        out_specs=[spec_out, spec_out],
        out_shape=[oshape, oshape],
    )(tu, tp)


def _fm_body(user_hbm, product_hbm, uf_hbm, pf_hbm, ub_hbm, pb_hbm, bias_hbm,
             out_hbm, idx_u2, idx_p2, idx_uf, idx_pf, ubuf, pbuf, ub_v, pb_v,
             bias_v, out_v, dummy, sem, sem2):
    wid = lax.axis_index("s") * _NC + lax.axis_index("c")
    base = wid * _BPW

    for j in range(_NCH):
        pltpu.sync_copy(user_hbm.at[pl.ds(base + j * _CH, _CH)], idx_u2.at[j])
        pltpu.sync_copy(product_hbm.at[pl.ds(base + j * _CH, _CH)], idx_p2.at[j])
    pltpu.sync_copy(user_hbm.at[pl.ds(base, _BPW)], idx_uf)
    pltpu.sync_copy(product_hbm.at[pl.ds(base, _BPW)], idx_pf)
    pltpu.sync_copy(bias_hbm, bias_v)

    bcopies = []
    for j in range(_NCH):
        sl = pl.ds(j * _CH, _CH)
        bcopies.append(pltpu.async_copy(ub_hbm.at[idx_u2.at[j]], ub_v.at[sl], sem2))
        bcopies.append(pltpu.async_copy(pb_hbm.at[idx_p2.at[j]], pb_v.at[sl], sem2))

    def fire_u(b, carry):
        r = idx_uf[pl.ds(b * _L, _L)]
        for k in range(_K):
            pltpu.async_copy(uf_hbm.at[r + (k * _STRIDE)],
                             ubuf.at[k, pl.ds(b * _L, _L)], sem)
        return carry

    def fire_p(b, carry):
        r = idx_pf[pl.ds(b * _L, _L)]
        for k in range(_K):
            pltpu.async_copy(pf_hbm.at[r + (k * _STRIDE)],
                             pbuf.at[k, pl.ds(b * _L, _L)], sem)
        return carry

    lax.fori_loop(0, _BLK, fire_u, 0)
    lax.fori_loop(0, _BLK, fire_p, 0)

    pltpu.make_async_copy(uf_hbm.at[pl.ds(0, 2 * _BPW * _K)], dummy, sem).wait()
    for c in bcopies:
        c.wait()

    bias_vec = bias_v[...]

    def block(b, carry):
        sl = pl.ds(b * _L, _L)
        acc = bias_vec + ub_v[sl] + pb_v[sl]
        for k in range(_K):
            acc = acc + ubuf[k, sl] * pbuf[k, sl]
        out_v[sl] = acc
        return carry

    lax.fori_loop(0, _BLK, block, 0)
    pltpu.sync_copy(out_v, out_hbm.at[pl.ds(base, _BPW)])


@jax.jit
def kernel(user, product, user_embeds, products_embeds, user_bias,
           products_bias, bias):
    uf4, pf4 = _detile(user_embeds.T, products_embeds.T)
    uf = uf4.reshape(_FLAT)
    pf = pf4.reshape(_FLAT)
    bias16 = jnp.broadcast_to(bias, (_L,))
    f = pl.kernel(
        _fm_body,
        out_type=jax.ShapeDtypeStruct((_B,), jnp.float32),
        mesh=plsc.VectorSubcoreMesh(core_axis_name="c", subcore_axis_name="s"),
        compiler_params=pltpu.CompilerParams(use_tc_tiling_on_sc=False,
                                             needs_layout_passes=False),
        scratch_types=[
            pltpu.VMEM((_NCH, _CH), jnp.int32),
            pltpu.VMEM((_NCH, _CH), jnp.int32),
            pltpu.VMEM((_BPW,), jnp.int32),
            pltpu.VMEM((_BPW,), jnp.int32),
            pltpu.VMEM((_K, _BPW), jnp.float32),
            pltpu.VMEM((_K, _BPW), jnp.float32),
            pltpu.VMEM((_BPW,), jnp.float32),
            pltpu.VMEM((_BPW,), jnp.float32),
            pltpu.VMEM((_L,), jnp.float32),
            pltpu.VMEM((_BPW,), jnp.float32),
            pltpu.VMEM((2 * _BPW * _K,), jnp.float32),
            pltpu.SemaphoreType.DMA,
            pltpu.SemaphoreType.DMA,
        ],
    )
    return f(user, product, uf, pf, user_bias.reshape(-1),
             products_bias.reshape(-1), bias16)


# speedup vs baseline: 5.3011x; 1.0013x over previous
import jax
import jax.numpy as jnp
from jax import lax
from jax.experimental import pallas as pl
from jax.experimental.pallas import tpu as pltpu
from jax.experimental.pallas import tpu_sc as plsc

_INFO = plsc.get_sparse_core_info()
_NC, _NS, _L = _INFO.num_cores, _INFO.num_subcores, _INFO.num_lanes
_NW = _NC * _NS
_B = 16384
_K = 16
_V = 1000000
_BPW = _B // _NW
_CH = 128
_NCH = _BPW // _CH
_BLK = _BPW // _L

_W = 65536
_GRID = -(-_V // _W)
_WB = _W // 128
_RBW = _GRID * _WB
_STRIDE = _RBW * 128
_FLAT = _K * _STRIDE


def _detile_body(x_ref, y_ref, ox_ref, oy_ref):
    ox_ref[...] = x_ref[...].reshape(2, 8, _WB, 128)
    oy_ref[...] = y_ref[...].reshape(2, 8, _WB, 128)


def _detile(tu, tp):
    spec_in = pl.BlockSpec((_K, _W), lambda c: (0, c))
    spec_out = pl.BlockSpec((2, 8, _WB, 128), lambda c: (0, 0, c, 0))
    oshape = jax.ShapeDtypeStruct((2, 8, _RBW, 128), jnp.float32)
    return pl.pallas_call(
        _detile_body,
        grid=(_GRID,),
        in_specs=[spec_in, spec_in],
        out_specs=[spec_out, spec_out],
        out_shape=[oshape, oshape],
    )(tu, tp)


def _fm_body(user_hbm, product_hbm, uf_hbm, pf_hbm, ub_hbm, pb_hbm, bias_hbm,
             out_hbm, idx_u2, idx_p2, idx_uf, idx_pf, ubuf, pbuf, ub_v, pb_v,
             bias_v, out_v, dummy, sem, sem2):
    wid = lax.axis_index("s") * _NC + lax.axis_index("c")
    base = wid * _BPW

    for j in range(_NCH):
        pltpu.sync_copy(user_hbm.at[pl.ds(base + j * _CH, _CH)], idx_u2.at[j])
        pltpu.sync_copy(product_hbm.at[pl.ds(base + j * _CH, _CH)], idx_p2.at[j])
    pltpu.sync_copy(user_hbm.at[pl.ds(base, _BPW)], idx_uf)
    pltpu.sync_copy(product_hbm.at[pl.ds(base, _BPW)], idx_pf)
    pltpu.sync_copy(bias_hbm, bias_v)

    bcopies = []
    for j in range(_NCH):
        sl = pl.ds(j * _CH, _CH)
        bcopies.append(pltpu.async_copy(ub_hbm.at[idx_u2.at[j]], ub_v.at[sl], sem2))
        bcopies.append(pltpu.async_copy(pb_hbm.at[idx_p2.at[j]], pb_v.at[sl], sem2))

    def fire_u(b, carry):
        r = idx_uf[pl.ds(b * _L, _L)]
        for k in range(_K):
            pltpu.async_copy(uf_hbm.at[r + (k * _STRIDE)],
                             ubuf.at[k, pl.ds(b * _L, _L)], sem)
        return carry

    def fire_p(b, carry):
        r = idx_pf[pl.ds(b * _L, _L)]
        for k in range(_K):
            pltpu.async_copy(pf_hbm.at[r + (k * _STRIDE)],
                             pbuf.at[k, pl.ds(b * _L, _L)], sem)
        return carry

    lax.fori_loop(0, _BLK, fire_u, 0)
    lax.fori_loop(0, _BLK, fire_p, 0)

    pltpu.make_async_copy(uf_hbm.at[pl.ds(0, 2 * _BPW * _K)], dummy, sem).wait()
    for c in bcopies:
        c.wait()

    bias_vec = bias_v[...]

    def block(b, carry):
        sl = pl.ds(b * _L, _L)
        acc = bias_vec + ub_v[sl] + pb_v[sl]
        for k in range(_K):
            acc = acc + ubuf[k, sl] * pbuf[k, sl]
        out_v[sl] = acc
        return carry

    lax.fori_loop(0, _BLK, block, 0)
    pltpu.sync_copy(out_v, out_hbm.at[pl.ds(base, _BPW)])


@jax.jit
def kernel(user, product, user_embeds, products_embeds, user_bias,
           products_bias, bias):
    uf4, pf4 = _detile(user_embeds.T, products_embeds.T)
    uf = uf4.reshape(_FLAT)
    pf = pf4.reshape(_FLAT)
    bias16 = jnp.broadcast_to(bias, (_L,))
    f = pl.kernel(
        _fm_body,
        out_type=jax.ShapeDtypeStruct((_B,), jnp.float32),
        mesh=plsc.VectorSubcoreMesh(core_axis_name="c", subcore_axis_name="s"),
        compiler_params=pltpu.CompilerParams(use_tc_tiling_on_sc=False,
                                             needs_layout_passes=False),
        scratch_types=[
            pltpu.VMEM((_NCH, _CH), jnp.int32),
            pltpu.VMEM((_NCH, _CH), jnp.int32),
            pltpu.VMEM((_BPW,), jnp.int32),
            pltpu.VMEM((_BPW,), jnp.int32),
            pltpu.VMEM((_K, _BPW), jnp.float32),
            pltpu.VMEM((_K, _BPW), jnp.float32),
            pltpu.VMEM((_BPW,), jnp.float32),
            pltpu.VMEM((_BPW,), jnp.float32),
            pltpu.VMEM((_L,), jnp.float32),
            pltpu.VMEM((_BPW,), jnp.float32),
            pltpu.VMEM((2 * _BPW * _K,), jnp.float32),
            pltpu.SemaphoreType.DMA,
            pltpu.SemaphoreType.DMA,
        ],
    )
    return f(user, product, uf, pf, user_bias.reshape(-1),
             products_bias.reshape(-1), bias16)


# speedup vs baseline: 6.1944x; 1.1685x over previous
import jax
import jax.numpy as jnp
from jax import lax
from jax.experimental import pallas as pl
from jax.experimental.pallas import tpu as pltpu
from jax.experimental.pallas import tpu_sc as plsc

_INFO = plsc.get_sparse_core_info()
_NC, _NS, _L = _INFO.num_cores, _INFO.num_subcores, _INFO.num_lanes
_NW = _NC * _NS
_B = 16384
_K = 16
_V = 1000000
_BPW = _B // _NW
_CH = 128
_NCH = _BPW // _CH
_BLK = _BPW // _L

_W = 32768
_GRID = -(-_V // _W)
_WB = _W // 128
_RBW = _GRID * _WB
_STRIDE = _RBW * 128
_FLAT = _K * _STRIDE


def _detile_body(x_ref, ox_ref):
    ox_ref[...] = x_ref[...].reshape(2, 8, _WB, 128)


def _detile_tc(tu):
    return pl.pallas_call(
        _detile_body,
        grid=(_GRID,),
        in_specs=[pl.BlockSpec((_K, _W), lambda c: (0, c))],
        out_specs=pl.BlockSpec((2, 8, _WB, 128), lambda c: (0, 0, c, 0)),
        out_shape=jax.ShapeDtypeStruct((2, 8, _RBW, 128), jnp.float32),
    )(tu)


_COLS = 7813
_RBW2 = 7816
_STRIDE2 = _RBW2 * 128
_FLAT2 = _K * _STRIDE2
_NWIN = 977
_WPW = 31
_VTAIL = _V - (_COLS - 1) * 128


def _sc_detile_body(t3_hbm, tail_hbm, o_hbm, vbuf, vtail, sem):
    wid = lax.axis_index("s") * _NC + lax.axis_index("c")
    w0 = wid * _WPW
    nw = jnp.minimum(_WPW, _NWIN - w0)

    def window(i, carry):
        c0 = (w0 + i) * 8
        for j in range(8):
            @pl.when(c0 + j < _COLS - 1)
            def _():
                sl = pl.ds((c0 + j) * 128, 128)
                pltpu.async_copy(t3_hbm.at[0, :, sl], vbuf.at[0, :, j], sem)
                pltpu.async_copy(t3_hbm.at[1, :, sl], vbuf.at[1, :, j], sem)
        for j in range(8):
            @pl.when(c0 + j < _COLS - 1)
            def _():
                sl = pl.ds((c0 + j) * 128, 128)
                pltpu.make_async_copy(t3_hbm.at[0, :, sl], vbuf.at[0, :, j], sem).wait()
                pltpu.make_async_copy(t3_hbm.at[1, :, sl], vbuf.at[1, :, j], sem).wait()
        pltpu.sync_copy(vbuf, o_hbm.at[:, :, pl.ds(c0, 8), :])
        return carry

    lax.fori_loop(0, nw, window, 0)

    @pl.when(wid == _NW - 1)
    def _():
        for kb in range(2):
            pltpu.sync_copy(tail_hbm.at[kb], vtail.at[kb, :, 0, :])
        pltpu.sync_copy(vtail, o_hbm.at[:, :, pl.ds(_COLS - 1, 1), :])


def _detile_sc(t3, tail):
    f = pl.kernel(
        _sc_detile_body,
        out_type=jax.ShapeDtypeStruct((2, 8, _RBW2, 128), jnp.float32),
        mesh=plsc.VectorSubcoreMesh(core_axis_name="c", subcore_axis_name="s"),
        compiler_params=pltpu.CompilerParams(use_tc_tiling_on_sc=True,
                                             needs_layout_passes=False),
        scratch_types=[
            pltpu.VMEM((2, 8, 8, 128), jnp.float32),
            pltpu.VMEM((2, 8, 1, 128), jnp.float32),
            pltpu.SemaphoreType.DMA,
        ],
    )
    return f(t3, tail)


def _fm_body(user_hbm, product_hbm, uf_hbm, pf_hbm, ub_hbm, pb_hbm, bias_hbm,
             out_hbm, idx_u2, idx_p2, idx_uf, idx_pf, ubuf, pbuf, ub_v, pb_v,
             bias_v, out_v, dummy, sem, sem2):
    wid = lax.axis_index("s") * _NC + lax.axis_index("c")
    base = wid * _BPW

    for j in range(_NCH):
        pltpu.sync_copy(user_hbm.at[pl.ds(base + j * _CH, _CH)], idx_u2.at[j])
        pltpu.sync_copy(product_hbm.at[pl.ds(base + j * _CH, _CH)], idx_p2.at[j])
    pltpu.sync_copy(user_hbm.at[pl.ds(base, _BPW)], idx_uf)
    pltpu.sync_copy(product_hbm.at[pl.ds(base, _BPW)], idx_pf)
    pltpu.sync_copy(bias_hbm, bias_v)

    bcopies = []
    for j in range(_NCH):
        sl = pl.ds(j * _CH, _CH)
        bcopies.append(pltpu.async_copy(ub_hbm.at[idx_u2.at[j]], ub_v.at[sl], sem2))
        bcopies.append(pltpu.async_copy(pb_hbm.at[idx_p2.at[j]], pb_v.at[sl], sem2))

    def fire_u(b, carry):
        r = idx_uf[pl.ds(b * _L, _L)]
        for k in range(_K):
            pltpu.async_copy(uf_hbm.at[r + (k * _STRIDE)],
                             ubuf.at[k, pl.ds(b * _L, _L)], sem)
        return carry

    def fire_p(b, carry):
        r = idx_pf[pl.ds(b * _L, _L)]
        for k in range(_K):
            pltpu.async_copy(pf_hbm.at[r + (k * _STRIDE2)],
                             pbuf.at[k, pl.ds(b * _L, _L)], sem)
        return carry

    lax.fori_loop(0, _BLK, fire_u, 0)
    lax.fori_loop(0, _BLK, fire_p, 0)

    pltpu.make_async_copy(uf_hbm.at[pl.ds(0, 2 * _BPW * _K)], dummy, sem).wait()
    for c in bcopies:
        c.wait()

    bias_vec = bias_v[...]

    def block(b, carry):
        sl = pl.ds(b * _L, _L)
        acc = bias_vec + ub_v[sl] + pb_v[sl]
        for k in range(_K):
            acc = acc + ubuf[k, sl] * pbuf[k, sl]
        out_v[sl] = acc
        return carry

    lax.fori_loop(0, _BLK, block, 0)
    pltpu.sync_copy(out_v, out_hbm.at[pl.ds(base, _BPW)])


@jax.jit
def kernel(user, product, user_embeds, products_embeds, user_bias,
           products_bias, bias):
    uf = _detile_tc(user_embeds.T).reshape(_FLAT)
    ptail = jnp.pad(products_embeds[_V - _VTAIL:].T,
                    ((0, 0), (0, 128 - _VTAIL))).reshape(2, 8, 128)
    pf = _detile_sc(products_embeds.T.reshape(2, 8, _V), ptail).reshape(_FLAT2)
    bias16 = jnp.broadcast_to(bias, (_L,))
    f = pl.kernel(
        _fm_body,
        out_type=jax.ShapeDtypeStruct((_B,), jnp.float32),
        mesh=plsc.VectorSubcoreMesh(core_axis_name="c", subcore_axis_name="s"),
        compiler_params=pltpu.CompilerParams(use_tc_tiling_on_sc=False,
                                             needs_layout_passes=False),
        scratch_types=[
            pltpu.VMEM((_NCH, _CH), jnp.int32),
            pltpu.VMEM((_NCH, _CH), jnp.int32),
            pltpu.VMEM((_BPW,), jnp.int32),
            pltpu.VMEM((_BPW,), jnp.int32),
            pltpu.VMEM((_K, _BPW), jnp.float32),
            pltpu.VMEM((_K, _BPW), jnp.float32),
            pltpu.VMEM((_BPW,), jnp.float32),
            pltpu.VMEM((_BPW,), jnp.float32),
            pltpu.VMEM((_L,), jnp.float32),
            pltpu.VMEM((_BPW,), jnp.float32),
            pltpu.VMEM((2 * _BPW * _K,), jnp.float32),
            pltpu.SemaphoreType.DMA,
            pltpu.SemaphoreType.DMA,
        ],
    )
    return f(user, product, uf, pf, user_bias.reshape(-1),
             products_bias.reshape(-1), bias16)
